# baseline (speedup 1.0000x reference)
import functools

import jax
import jax.numpy as jnp
from jax import lax
from jax.experimental import pallas as pl
from jax.experimental.pallas import tpu as pltpu
from jax.experimental.pallas import tpu_sc as plsc

_NUM_ATTRS = 26
_VOCAB = 100000
_EMBED_DIM = 32
_BATCH = 16384

_TOTAL = _BATCH * _NUM_ATTRS
_G = 128
_NGROUPS = _TOTAL // _G
_NW = 32
_GPW = _NGROUPS // _NW
_K = 8
_NSTEPS = _GPW // _K

_mesh = plsc.VectorSubcoreMesh(core_axis_name="c", subcore_axis_name="s")


@functools.partial(
    pl.kernel,
    out_type=jax.ShapeDtypeStruct((_NGROUPS, _G, _EMBED_DIM), jnp.float32),
    mesh=_mesh,
    scratch_types=[
        pltpu.VMEM((_GPW, _G), jnp.int32),
        pltpu.VMEM((2, _K, _G, _EMBED_DIM), jnp.float32),
        pltpu.SemaphoreType.DMA((2,)),
        pltpu.SemaphoreType.DMA((2,)),
    ],
    compiler_params=pltpu.CompilerParams(use_tc_tiling_on_sc=False),
)
def _lookup(tup_hbm, tab_hbm, out_hbm, idx_v, rows_v, gsem, osem):
  wid = lax.axis_index("s") * 2 + lax.axis_index("c")
  gbase = wid * _GPW

  pltpu.sync_copy(tup_hbm.at[pl.ds(gbase, _GPW)], idx_v)

  def fix_row(r, carry):
    p0 = (gbase + r) * _G
    for c in range(_G // 16):
      pos = p0 + c * 16 + lax.iota(jnp.int32, 16)
      off = lax.rem(pos, _NUM_ATTRS) * _VOCAB
      idx_v[r, pl.ds(c * 16, 16)] = idx_v[r, pl.ds(c * 16, 16)] + off
    return carry

  lax.fori_loop(0, _GPW, fix_row, 0)

  def fire(step, slot):
    for j in range(_K):
      pltpu.async_copy(
          tab_hbm.at[idx_v.at[step * _K + j]], rows_v.at[slot, j],
          gsem.at[slot])

  def drain_gather(slot):
    pltpu.make_async_copy(
        out_hbm.at[pl.ds(0, _K)], rows_v.at[slot], gsem.at[slot]).wait()

  def drain_out(slot):
    pltpu.make_async_copy(
        rows_v.at[slot], out_hbm.at[pl.ds(0, _K)], osem.at[slot]).wait()

  fire(0, 0)

  def step_body(s, carry):
    slot = lax.rem(s, 2)
    drain_gather(slot)

    @pl.when(s + 1 < _NSTEPS)
    def _prefetch():
      @pl.when(s >= 1)
      def _():
        drain_out(1 - slot)
      fire(s + 1, 1 - slot)

    pltpu.async_copy(
        rows_v.at[slot], out_hbm.at[pl.ds(gbase + s * _K, _K)], osem.at[slot])
    return carry

  lax.fori_loop(0, _NSTEPS, step_body, 0)
  drain_out(0)
  drain_out(1)


def kernel(tuples, tables):
  tup = tuples.astype(jnp.int32).reshape(_NGROUPS, _G)
  tab = tables.reshape(_NUM_ATTRS * _VOCAB, _EMBED_DIM)
  out = _lookup(tup, tab)
  return out.reshape(_BATCH, _NUM_ATTRS * _EMBED_DIM)


# speedup vs baseline: 3.2406x; 3.2406x over previous
import functools

import jax
import jax.numpy as jnp
from jax import lax
from jax.experimental import pallas as pl
from jax.experimental.pallas import tpu as pltpu
from jax.experimental.pallas import tpu_sc as plsc

_NUM_ATTRS = 26
_VOCAB = 100000
_EMBED_DIM = 32
_BATCH = 16384

_HALF = _BATCH // 2

_mesh = plsc.VectorSubcoreMesh(core_axis_name="c", subcore_axis_name="s")


@functools.partial(
    pl.kernel,
    out_type=jax.ShapeDtypeStruct((_NUM_ATTRS * _EMBED_DIM, _BATCH),
                                  jnp.float32),
    mesh=_mesh,
    scratch_types=[
        pltpu.VMEM((_VOCAB,), jnp.float32),
        pltpu.VMEM((_BATCH,), jnp.int32),
        pltpu.VMEM((_HALF,), jnp.float32),
    ],
    compiler_params=pltpu.CompilerParams(needs_layout_passes=False),
)
def _lookup(tup_hbm, tab_hbm, out_hbm, row_v, idx_v, outb_v):
  e = lax.axis_index("c") * 16 + lax.axis_index("s")

  def per_attr(a, carry):
    pltpu.sync_copy(tup_hbm.at[a], idx_v)
    pltpu.sync_copy(tab_hbm.at[a, e], row_v)
    r = a * _EMBED_DIM + e

    def half(h):
      def g(k, c2):
        iv = idx_v[pl.ds(h * _HALF + k * 16, 16)]
        outb_v[pl.ds(k * 16, 16)] = plsc.load_gather(row_v, [iv])
        return c2

      lax.fori_loop(0, _HALF // 16, g, 0)
      pltpu.sync_copy(outb_v, out_hbm.at[r, pl.ds(h * _HALF, _HALF)])

    half(0)
    half(1)
    return carry

  lax.fori_loop(0, _NUM_ATTRS, per_attr, 0)


def kernel(tuples, tables):
  tup_t = tuples.astype(jnp.int32).T
  tab_t = jnp.transpose(tables, (0, 2, 1))
  out_t = _lookup(tup_t, tab_t)
  return out_t.T


# speedup vs baseline: 6.6280x; 2.0453x over previous
import functools

import jax
import jax.numpy as jnp
from jax import lax
from jax.experimental import pallas as pl
from jax.experimental.pallas import tpu as pltpu
from jax.experimental.pallas import tpu_sc as plsc

_NUM_ATTRS = 26
_VOCAB = 100000
_EMBED_DIM = 32
_BATCH = 16384

_NW = 32
_ROWS_PER_W = _NUM_ATTRS * _EMBED_DIM // _NW
_Q = 4096
_NQ = _BATCH // _Q

_mesh = plsc.VectorSubcoreMesh(core_axis_name="c", subcore_axis_name="s")


@functools.partial(
    pl.kernel,
    out_type=jax.ShapeDtypeStruct((_NUM_ATTRS * _EMBED_DIM, _BATCH),
                                  jnp.float32),
    mesh=_mesh,
    scratch_types=[
        pltpu.VMEM((_VOCAB,), jnp.float32),
        pltpu.VMEM((_BATCH,), jnp.int32),
        pltpu.VMEM((2, _Q), jnp.float32),
        pltpu.SemaphoreType.DMA((2,)),
    ],
    compiler_params=pltpu.CompilerParams(needs_layout_passes=False),
)
def _lookup(tup_hbm, tab_hbm, out_hbm, row_v, idx_v, outb_v, osem):
  w = lax.axis_index("s") * 2 + lax.axis_index("c")
  r0 = w * _ROWS_PER_W

  def per_row(i, prev_a):
    r = r0 + i
    a = r // _EMBED_DIM
    e = r % _EMBED_DIM

    @pl.when(a != prev_a)
    def _():
      pltpu.sync_copy(tup_hbm.at[a], idx_v)

    pltpu.sync_copy(tab_hbm.at[a, e], row_v)

    for q in range(_NQ):
      slot = q % 2

      @pl.when(i * _NQ + q >= 2)
      def _():
        pltpu.make_async_copy(
            outb_v.at[slot], out_hbm.at[r, pl.ds(0, _Q)], osem.at[slot]
        ).wait()

      @plsc.parallel_loop(0, _Q // 16, 1, unroll=8)
      def _(k):
        iv = idx_v[pl.ds(q * _Q + k * 16, 16)]
        outb_v[slot, pl.ds(k * 16, 16)] = plsc.load_gather(row_v, [iv])

      pltpu.async_copy(
          outb_v.at[slot], out_hbm.at[r, pl.ds(q * _Q, _Q)], osem.at[slot])
    return a

  lax.fori_loop(0, _ROWS_PER_W, per_row, -1)
  for slot in range(2):
    pltpu.make_async_copy(
        outb_v.at[slot], out_hbm.at[0, pl.ds(0, _Q)], osem.at[slot]).wait()


def kernel(tuples, tables):
  tup_t = tuples.astype(jnp.int32).T
  tab_t = jnp.transpose(tables, (0, 2, 1))
  out_t = _lookup(tup_t, tab_t)
  return out_t.T
